# baseline (device time: 12610 ns/iter reference)
import jax
import jax.numpy as jnp
from jax import lax
from jax.experimental import pallas as pl
from jax.experimental.pallas import tpu as pltpu


def kernel(x, dy, gamma):
    m, d = x.shape
    half = m // 2

    def body(x_hbm, dy_hbm, gamma_ref, out_ref,
             xv, dyv, comm, local_sems, send_sems, recv_sems):
        my_x = lax.axis_index("x")
        my_y = lax.axis_index("y")
        x_nbr = (1 - my_x, my_y)
        y_nbr = (my_x, 1 - my_y)

        barrier_sem = pltpu.get_barrier_semaphore()
        for nbr in (x_nbr, y_nbr):
            pl.semaphore_signal(
                barrier_sem, inc=1, device_id=nbr,
                device_id_type=pl.DeviceIdType.MESH,
            )

        off = my_x * half
        cp_x = pltpu.make_async_copy(
            x_hbm.at[pl.ds(off, half), :], xv, local_sems.at[0])
        cp_dy = pltpu.make_async_copy(
            dy_hbm.at[pl.ds(off, half), :], dyv, local_sems.at[1])
        cp_x.start()
        cp_dy.start()
        cp_x.wait()
        cp_dy.wait()

        xf = xv[:, :].astype(jnp.float32)
        dyf = dyv[:, :].astype(jnp.float32)
        mu = jnp.mean(xf, axis=1, keepdims=True)
        xc = xf - mu
        var = jnp.mean(xc * xc, axis=1, keepdims=True)
        rstd = lax.rsqrt(var + 1e-5)
        dgamma = jnp.sum(dyf * (xc * rstd), axis=0)
        dbeta = jnp.sum(dyf, axis=0)
        comm[0, :, :] = jnp.stack([dgamma, dbeta])

        pl.semaphore_wait(barrier_sem, 2)

        rdma0 = pltpu.make_async_remote_copy(
            src_ref=comm.at[0], dst_ref=comm.at[1],
            send_sem=send_sems.at[0], recv_sem=recv_sems.at[0],
            device_id=x_nbr, device_id_type=pl.DeviceIdType.MESH,
        )
        rdma0.start()
        rdma0.wait()
        comm[0, :, :] = comm[0, :, :] + comm[1, :, :]

        rdma1 = pltpu.make_async_remote_copy(
            src_ref=comm.at[0], dst_ref=comm.at[2],
            send_sem=send_sems.at[1], recv_sem=recv_sems.at[1],
            device_id=y_nbr, device_id_type=pl.DeviceIdType.MESH,
        )
        rdma1.start()
        rdma1.wait()
        out_ref[:, :] = comm[0, :, :] + comm[2, :, :]

    return pl.pallas_call(
        body,
        out_shape=jax.ShapeDtypeStruct((2, d), jnp.float32),
        in_specs=[
            pl.BlockSpec(memory_space=pltpu.MemorySpace.HBM),
            pl.BlockSpec(memory_space=pltpu.MemorySpace.HBM),
            pl.BlockSpec(memory_space=pltpu.MemorySpace.VMEM),
        ],
        out_specs=pl.BlockSpec(memory_space=pltpu.MemorySpace.VMEM),
        scratch_shapes=[
            pltpu.VMEM((half, d), jnp.float32),
            pltpu.VMEM((half, d), jnp.float32),
            pltpu.VMEM((3, 2, d), jnp.float32),
            pltpu.SemaphoreType.DMA((2,)),
            pltpu.SemaphoreType.DMA((2,)),
            pltpu.SemaphoreType.DMA((2,)),
        ],
        compiler_params=pltpu.CompilerParams(collective_id=0),
    )(x, dy, gamma)


# device time: 11420 ns/iter; 1.1042x vs baseline; 1.1042x over previous
import jax
import jax.numpy as jnp
from jax import lax
from jax.experimental import pallas as pl
from jax.experimental.pallas import tpu as pltpu


def kernel(x, dy, gamma):
    m, d = x.shape
    half = m // 2

    def body(x_hbm, dy_hbm, gamma_ref, out_ref,
             xv, dyv, comm, local_sems, send_sems, recv_sems):
        my_x = lax.axis_index("x")
        my_y = lax.axis_index("y")
        my_sid = my_x * 2 + my_y
        peers = [(1 - my_x, my_y), (my_x, 1 - my_y), (1 - my_x, 1 - my_y)]

        barrier_sem = pltpu.get_barrier_semaphore()
        for nbr in peers:
            pl.semaphore_signal(
                barrier_sem, inc=1, device_id=nbr,
                device_id_type=pl.DeviceIdType.MESH,
            )

        off = my_x * half
        cp_x = pltpu.make_async_copy(
            x_hbm.at[pl.ds(off, half), :], xv, local_sems.at[0])
        cp_dy = pltpu.make_async_copy(
            dy_hbm.at[pl.ds(off, half), :], dyv, local_sems.at[1])
        cp_x.start()
        cp_dy.start()
        cp_x.wait()
        cp_dy.wait()

        xf = xv[:, :].astype(jnp.float32)
        dyf = dyv[:, :].astype(jnp.float32)
        mu = jnp.mean(xf, axis=1, keepdims=True)
        xc = xf - mu
        var = jnp.mean(xc * xc, axis=1, keepdims=True)
        rstd = lax.rsqrt(var + 1e-5)
        dgamma = jnp.sum(dyf * (xc * rstd), axis=0)
        dbeta = jnp.sum(dyf, axis=0)
        comm[my_sid, :, :] = jnp.stack([dgamma, dbeta])

        pl.semaphore_wait(barrier_sem, 3)

        rdmas = []
        for i, nbr in enumerate(peers):
            r = pltpu.make_async_remote_copy(
                src_ref=comm.at[my_sid], dst_ref=comm.at[my_sid],
                send_sem=send_sems.at[i], recv_sem=recv_sems.at[my_sid],
                device_id=nbr, device_id_type=pl.DeviceIdType.MESH,
            )
            r.start()
            rdmas.append(r)

        for nbr in peers:
            p_sid = nbr[0] * 2 + nbr[1]
            pltpu.make_async_remote_copy(
                src_ref=comm.at[p_sid], dst_ref=comm.at[p_sid],
                send_sem=send_sems.at[0], recv_sem=recv_sems.at[p_sid],
                device_id=nbr, device_id_type=pl.DeviceIdType.MESH,
            ).wait_recv()

        out_ref[:, :] = (
            (comm[0, :, :] + comm[1, :, :]) + (comm[2, :, :] + comm[3, :, :])
        )

        for r in rdmas:
            r.wait_send()

    return pl.pallas_call(
        body,
        out_shape=jax.ShapeDtypeStruct((2, d), jnp.float32),
        in_specs=[
            pl.BlockSpec(memory_space=pltpu.MemorySpace.HBM),
            pl.BlockSpec(memory_space=pltpu.MemorySpace.HBM),
            pl.BlockSpec(memory_space=pltpu.MemorySpace.VMEM),
        ],
        out_specs=pl.BlockSpec(memory_space=pltpu.MemorySpace.VMEM),
        scratch_shapes=[
            pltpu.VMEM((half, d), jnp.float32),
            pltpu.VMEM((half, d), jnp.float32),
            pltpu.VMEM((4, 2, d), jnp.float32),
            pltpu.SemaphoreType.DMA((2,)),
            pltpu.SemaphoreType.DMA((3,)),
            pltpu.SemaphoreType.DMA((4,)),
        ],
        compiler_params=pltpu.CompilerParams(collective_id=0),
    )(x, dy, gamma)


# device time: 10600 ns/iter; 1.1896x vs baseline; 1.0774x over previous
import jax
import jax.numpy as jnp
from jax import lax
from jax.experimental import pallas as pl
from jax.experimental.pallas import tpu as pltpu


def kernel(x, dy, gamma):
    m, d = x.shape

    def body(x_ref, dy_ref, gamma_ref, out_ref, comm_ref, send_sem, recv_sem):
        my_x = lax.axis_index("x")
        my_y = lax.axis_index("y")
        nbr = (my_x, 1 - my_y)

        barrier_sem = pltpu.get_barrier_semaphore()
        pl.semaphore_signal(
            barrier_sem, inc=1, device_id=nbr,
            device_id_type=pl.DeviceIdType.MESH,
        )

        xv = x_ref[:, :].astype(jnp.float32)
        dyv = dy_ref[:, :].astype(jnp.float32)
        mu = jnp.mean(xv, axis=1, keepdims=True)
        xc = xv - mu
        var = jnp.mean(xc * xc, axis=1, keepdims=True)
        rstd = lax.rsqrt(var + 1e-5)
        dgamma = jnp.sum(dyv * (xc * rstd), axis=0)
        dbeta = jnp.sum(dyv, axis=0)
        comm_ref[0, :, :] = jnp.stack([dgamma, dbeta])

        pl.semaphore_wait(barrier_sem, 1)

        rdma = pltpu.make_async_remote_copy(
            src_ref=comm_ref.at[0],
            dst_ref=comm_ref.at[1],
            send_sem=send_sem,
            recv_sem=recv_sem,
            device_id=nbr,
            device_id_type=pl.DeviceIdType.MESH,
        )
        rdma.start()
        rdma.wait()

        out_ref[:, :] = comm_ref[0, :, :] + comm_ref[1, :, :]

    return pl.pallas_call(
        body,
        out_shape=jax.ShapeDtypeStruct((2, d), jnp.float32),
        in_specs=[
            pl.BlockSpec(memory_space=pltpu.VMEM),
            pl.BlockSpec(memory_space=pltpu.VMEM),
            pl.BlockSpec(memory_space=pltpu.VMEM),
        ],
        out_specs=pl.BlockSpec(memory_space=pltpu.VMEM),
        scratch_shapes=[
            pltpu.VMEM((2, 2, d), jnp.float32),
            pltpu.SemaphoreType.DMA,
            pltpu.SemaphoreType.DMA,
        ],
        compiler_params=pltpu.CompilerParams(collective_id=0),
    )(x, dy, gamma)
